# Initial kernel scaffold; baseline (speedup 1.0000x reference)
import jax
import jax.numpy as jnp
from jax.experimental import pallas as pl


def kernel(x, M):
    raise NotImplementedError("write your pallas kernel here")


# speedup vs baseline: 6.0021x; 6.0021x over previous
import jax
import jax.numpy as jnp
from jax.experimental import pallas as pl


def _matmul_block(x_ref, m_ref, o_ref):
    o_ref[...] = jax.lax.dot_general(
        x_ref[...],
        m_ref[...],
        dimension_numbers=(((1,), (1,)), ((), ())),
        preferred_element_type=jnp.float32,
    )


def kernel(x, M):
    B, C, N = x.shape
    O = M.shape[0]
    BC = B * C
    x2 = x.reshape(BC, N)

    OT = 128
    y = pl.pallas_call(
        _matmul_block,
        grid=(pl.cdiv(O, OT),),
        in_specs=[
            pl.BlockSpec((BC, N), lambda i: (0, 0)),
            pl.BlockSpec((OT, N), lambda i: (i, 0)),
        ],
        out_specs=pl.BlockSpec((BC, OT), lambda i: (0, i)),
        out_shape=jax.ShapeDtypeStruct((BC, O), jnp.float32),
    )(x2, M)
    return y.reshape(B, C, O)


# speedup vs baseline: 6.8502x; 1.1413x over previous
import jax
import jax.numpy as jnp
from jax.experimental import pallas as pl


def _matmul_block(x_ref, m_ref, o_ref):
    o_ref[...] = jax.lax.dot_general(
        x_ref[...],
        m_ref[...],
        dimension_numbers=(((1,), (1,)), ((), ())),
        preferred_element_type=jnp.float32,
    )


def kernel(x, M):
    B, C, N = x.shape
    O = M.shape[0]
    BC = B * C
    x2 = x.reshape(BC, N)

    OT = 256
    y = pl.pallas_call(
        _matmul_block,
        grid=(pl.cdiv(O, OT),),
        in_specs=[
            pl.BlockSpec((BC, N), lambda i: (0, 0)),
            pl.BlockSpec((OT, N), lambda i: (i, 0)),
        ],
        out_specs=pl.BlockSpec((BC, OT), lambda i: (0, i)),
        out_shape=jax.ShapeDtypeStruct((BC, O), jnp.float32),
    )(x2, M)
    return y.reshape(B, C, O)
